# baseline (speedup 1.0000x reference)
import functools

import jax
import jax.numpy as jnp
from jax.experimental import pallas as pl
from jax.experimental.pallas import tpu as pltpu

_INV_SQRT2 = 0.7071067811865476


def _round_up(x, m):
    return (x + m - 1) // m * m


def _fold_kernel(ept_ref, ex_ref, ez_ref, ed_ref, w1_ref, istd_ref, b1_ref,
                 w2_ref, a1_ref, a2_ref, w2s_ref, *, dims):
    (n_pt, d_pt, n_x, d_x, n_z, d_z, n_d, d_d, n_c, L1, L2, RB, D) = dims
    w1 = w1_ref[...]
    r0 = d_pt
    r1 = d_pt + d_x
    r2 = d_pt + d_x + d_z
    r3 = d_pt + d_x + d_z + d_d
    a_pt = jnp.dot(ept_ref[...], w1[0:r0, :],
                   preferred_element_type=jnp.float32) + b1_ref[...]
    a_x = jnp.dot(ex_ref[...], w1[r0:r1, :],
                  preferred_element_type=jnp.float32)
    a_z = jnp.dot(ez_ref[...], w1[r1:r2, :],
                  preferred_element_type=jnp.float32)
    a_d = jnp.dot(ed_ref[...], w1[r2:r3, :],
                  preferred_element_type=jnp.float32)
    a_c = w1[r3:r3 + n_c, :] * istd_ref[...]
    parts1 = [a_pt, a_c]
    if L1 > n_pt + n_c:
        parts1.append(jnp.zeros((L1 - n_pt - n_c, D), jnp.float32))
    a1_ref[...] = jnp.concatenate(parts1, axis=0) * _INV_SQRT2
    pad2 = jnp.zeros((L2 - n_x - n_z - n_d, D), jnp.float32)
    a2_ref[...] = jnp.concatenate([a_x, a_z, a_d, pad2], axis=0) * _INV_SQRT2
    w2s_ref[...] = w2_ref[...] * (0.5 * 2.0 ** 0.5)


def _encoder_kernel(pt_ref, x_ref, z_ref, d_ref, cont_ref,
                    a1_ref, a2_ref, w2_ref, b2_ref,
                    mean_ref, istd_ref, out_ref, buf_ref, sem, *, dims):
    (n_pt, d_pt, n_x, d_x, n_z, d_z, n_d, d_d, n_c, L1, L2, RB, D) = dims
    S = pt_ref.shape[1]

    iota1 = jax.lax.broadcasted_iota(jnp.int32, (RB, S, n_pt), 2)
    oh_pt = (iota1 == pt_ref[...][:, :, None]).astype(jnp.float32)
    cont_n = (cont_ref[...] - mean_ref[0, 0, :]) * istd_ref[0, 0, :]
    aug1 = jnp.concatenate([oh_pt, cont_n], axis=2)

    iota2 = jax.lax.broadcasted_iota(jnp.int32, (RB, S, L2), 2)
    aug2 = ((iota2 == x_ref[...][:, :, None])
            | (iota2 == z_ref[...][:, :, None] + n_x)
            | (iota2 == d_ref[...][:, :, None] + n_x + n_z)
            ).astype(jnp.float32)

    dn = (((2,), (0,)), ((), ()))
    h = jax.lax.dot_general(aug1, a1_ref[...], dn,
                            preferred_element_type=jnp.float32)
    h += jax.lax.dot_general(aug2, a2_ref[...], dn,
                             preferred_element_type=jnp.float32)
    h = h * (1.0 + jax.lax.erf(h))
    out = jax.lax.dot_general(h, w2_ref[...], dn,
                              preferred_element_type=jnp.float32)
    i = pl.program_id(0)
    ng = pl.num_programs(0)
    slot = jax.lax.rem(i, 2)

    @pl.when(i >= 2)
    def _wait_prev():
        pltpu.make_async_copy(buf_ref.at[slot], out_ref.at[pl.ds(0, RB)],
                              sem.at[slot]).wait()

    buf_ref[slot] = out + b2_ref[0, 0, :]
    cp = pltpu.make_async_copy(
        buf_ref.at[slot], out_ref.at[pl.ds(i * RB, RB)], sem.at[slot])
    cp.start()

    @pl.when(i == ng - 1)
    def _drain():
        pltpu.make_async_copy(buf_ref.at[slot],
                              out_ref.at[pl.ds(i * RB, RB)],
                              sem.at[slot]).wait()

        @pl.when(i >= 1)
        def _drain2():
            other = 1 - slot
            pltpu.make_async_copy(buf_ref.at[other],
                                  out_ref.at[pl.ds(0, RB)],
                                  sem.at[other]).wait()


@jax.jit
def kernel(pitch_type_id, x_bin, z_bin, description_id, cont,
           emb_pitch_type, emb_x, emb_z, emb_desc,
           W1, b1, W2, b2, cont_mean, cont_std):
    B, S = pitch_type_id.shape
    n_pt, d_pt = emb_pitch_type.shape
    n_x, d_x = emb_x.shape
    n_z, d_z = emb_z.shape
    n_d, d_d = emb_desc.shape
    n_c = cont.shape[-1]
    D = W2.shape[1]
    L1 = _round_up(n_pt + n_c, 8)
    L2 = _round_up(n_x + n_z + n_d, 8)

    RB = 16
    assert B % RB == 0
    G = B // RB

    istd = 1.0 / jnp.clip(cont_std, 1e-6, None)
    dims = (n_pt, d_pt, n_x, d_x, n_z, d_z, n_d, d_d, n_c, L1, L2, RB, D)
    full = lambda shape: pl.BlockSpec(shape, lambda i: tuple(0 for _ in shape))

    a1, a2, w2s = pl.pallas_call(
        functools.partial(_fold_kernel, dims=dims),
        out_shape=(jax.ShapeDtypeStruct((L1, D), jnp.float32),
                   jax.ShapeDtypeStruct((L2, D), jnp.float32),
                   jax.ShapeDtypeStruct((D, D), jnp.float32)),
    )(emb_pitch_type, emb_x, emb_z, emb_desc, W1, istd.reshape(n_c, 1),
      b1.reshape(1, D), W2)

    out = pl.pallas_call(
        functools.partial(_encoder_kernel, dims=dims),
        grid=(G,),
        in_specs=[
            pl.BlockSpec((RB, S), lambda i: (i, 0)),
            pl.BlockSpec((RB, S), lambda i: (i, 0)),
            pl.BlockSpec((RB, S), lambda i: (i, 0)),
            pl.BlockSpec((RB, S), lambda i: (i, 0)),
            pl.BlockSpec((RB, S, n_c), lambda i: (i, 0, 0)),
            full((L1, D)),
            full((L2, D)),
            full((D, D)),
            pl.BlockSpec((1, 1, D), lambda i: (0, 0, 0)),
            pl.BlockSpec((1, 1, n_c), lambda i: (0, 0, 0)),
            pl.BlockSpec((1, 1, n_c), lambda i: (0, 0, 0)),
        ],
        out_specs=pl.BlockSpec(memory_space=pltpu.MemorySpace.HBM),
        out_shape=jax.ShapeDtypeStruct((B, S, D), jnp.float32),
        scratch_shapes=[pltpu.VMEM((2, RB, S, D), jnp.float32),
                        pltpu.SemaphoreType.DMA((2,))],
        compiler_params=pltpu.CompilerParams(
            dimension_semantics=("arbitrary",)),
    )(pitch_type_id.astype(jnp.int32), x_bin.astype(jnp.int32),
      z_bin.astype(jnp.int32), description_id.astype(jnp.int32), cont,
      a1, a2, w2s, b2.reshape(1, 1, D),
      cont_mean.reshape(1, 1, n_c), istd.reshape(1, 1, n_c))

    return out


# speedup vs baseline: 1.1130x; 1.1130x over previous
import functools

import jax
import jax.numpy as jnp
from jax.experimental import pallas as pl
from jax.experimental.pallas import tpu as pltpu

_INV_SQRT2 = 0.7071067811865476


def _round_up(x, m):
    return (x + m - 1) // m * m


def _fold_kernel(ept_ref, ex_ref, ez_ref, ed_ref, w1_ref, istd_ref, b1_ref,
                 a1_ref, a2_ref, *, dims):
    (n_pt, d_pt, n_x, d_x, n_z, d_z, n_d, d_d, n_c, L1, L2, RB, D) = dims
    w1 = w1_ref[...]
    r0 = d_pt
    r1 = d_pt + d_x
    r2 = d_pt + d_x + d_z
    r3 = d_pt + d_x + d_z + d_d
    a_pt = jnp.dot(ept_ref[...], w1[0:r0, :],
                   preferred_element_type=jnp.float32) + b1_ref[...]
    a_x = jnp.dot(ex_ref[...], w1[r0:r1, :],
                  preferred_element_type=jnp.float32)
    a_z = jnp.dot(ez_ref[...], w1[r1:r2, :],
                  preferred_element_type=jnp.float32)
    a_d = jnp.dot(ed_ref[...], w1[r2:r3, :],
                  preferred_element_type=jnp.float32)
    a_c = w1[r3:r3 + n_c, :] * istd_ref[...]
    parts1 = [a_pt, a_c]
    if L1 > n_pt + n_c:
        parts1.append(jnp.zeros((L1 - n_pt - n_c, D), jnp.float32))
    a1_ref[...] = jnp.concatenate(parts1, axis=0)
    pad2 = jnp.zeros((L2 - n_x - n_z - n_d, D), jnp.float32)
    a2_ref[...] = jnp.concatenate([a_x, a_z, a_d, pad2], axis=0)


def _encoder_kernel(pt_ref, x_ref, z_ref, d_ref, cont_ref,
                    a1_ref, a2_ref, w2_ref, b2_ref,
                    mean_ref, istd_ref, out_ref, *, dims):
    (n_pt, d_pt, n_x, d_x, n_z, d_z, n_d, d_d, n_c, L1, L2, RB, D) = dims
    S = pt_ref.shape[1]

    iota1 = jax.lax.broadcasted_iota(jnp.int32, (RB, S, n_pt), 2)
    oh_pt = (iota1 == pt_ref[...][:, :, None]).astype(jnp.float32)
    cont_n = (cont_ref[...] - mean_ref[0, 0, :]) * istd_ref[0, 0, :]
    aug1 = jnp.concatenate([oh_pt, cont_n], axis=2)

    iota2 = jax.lax.broadcasted_iota(jnp.int32, (RB, S, L2), 2)
    aug2 = ((iota2 == x_ref[...][:, :, None])
            | (iota2 == z_ref[...][:, :, None] + n_x)
            | (iota2 == d_ref[...][:, :, None] + n_x + n_z)
            ).astype(jnp.float32)

    dn = (((2,), (0,)), ((), ()))
    h = jax.lax.dot_general(aug1, a1_ref[...], dn,
                            preferred_element_type=jnp.float32)
    h += jax.lax.dot_general(aug2, a2_ref[...], dn,
                             preferred_element_type=jnp.float32)
    h = 0.5 * h * (1.0 + jax.lax.erf(h * _INV_SQRT2))
    out = jax.lax.dot_general(h, w2_ref[...], dn,
                              preferred_element_type=jnp.float32)
    out_ref[...] = out + b2_ref[0, 0, :]


@jax.jit
def kernel(pitch_type_id, x_bin, z_bin, description_id, cont,
           emb_pitch_type, emb_x, emb_z, emb_desc,
           W1, b1, W2, b2, cont_mean, cont_std):
    B, S = pitch_type_id.shape
    n_pt, d_pt = emb_pitch_type.shape
    n_x, d_x = emb_x.shape
    n_z, d_z = emb_z.shape
    n_d, d_d = emb_desc.shape
    n_c = cont.shape[-1]
    D = W2.shape[1]
    L1 = _round_up(n_pt + n_c, 8)
    L2 = _round_up(n_x + n_z + n_d, 8)

    RB = 32
    assert B % RB == 0
    G = B // RB

    istd = 1.0 / jnp.clip(cont_std, 1e-6, None)
    dims = (n_pt, d_pt, n_x, d_x, n_z, d_z, n_d, d_d, n_c, L1, L2, RB, D)
    full = lambda shape: pl.BlockSpec(shape, lambda i: tuple(0 for _ in shape))

    a1, a2 = pl.pallas_call(
        functools.partial(_fold_kernel, dims=dims),
        out_shape=(jax.ShapeDtypeStruct((L1, D), jnp.float32),
                   jax.ShapeDtypeStruct((L2, D), jnp.float32)),
    )(emb_pitch_type, emb_x, emb_z, emb_desc, W1, istd.reshape(n_c, 1),
      b1.reshape(1, D))

    out = pl.pallas_call(
        functools.partial(_encoder_kernel, dims=dims),
        grid=(G,),
        in_specs=[
            pl.BlockSpec((RB, S), lambda i: (i, 0)),
            pl.BlockSpec((RB, S), lambda i: (i, 0)),
            pl.BlockSpec((RB, S), lambda i: (i, 0)),
            pl.BlockSpec((RB, S), lambda i: (i, 0)),
            pl.BlockSpec((RB, S, n_c), lambda i: (i, 0, 0)),
            full((L1, D)),
            full((L2, D)),
            full((D, D)),
            pl.BlockSpec((1, 1, D), lambda i: (0, 0, 0)),
            pl.BlockSpec((1, 1, n_c), lambda i: (0, 0, 0)),
            pl.BlockSpec((1, 1, n_c), lambda i: (0, 0, 0)),
        ],
        out_specs=pl.BlockSpec((RB, S, D), lambda i: (i, 0, 0)),
        out_shape=jax.ShapeDtypeStruct((B, S, D), jnp.float32),
        compiler_params=pltpu.CompilerParams(
            dimension_semantics=("parallel",)),
    )(pitch_type_id.astype(jnp.int32), x_bin.astype(jnp.int32),
      z_bin.astype(jnp.int32), description_id.astype(jnp.int32), cont,
      a1, a2, W2, b2.reshape(1, 1, D),
      cont_mean.reshape(1, 1, n_c), istd.reshape(1, 1, n_c))

    return out


# speedup vs baseline: 1.1511x; 1.0343x over previous
import functools

import jax
import jax.numpy as jnp
from jax.experimental import pallas as pl
from jax.experimental.pallas import tpu as pltpu

_INV_SQRT2 = 0.7071067811865476


def _round_up(x, m):
    return (x + m - 1) // m * m


def _fold_kernel(ept_ref, ex_ref, ez_ref, ed_ref, w1_ref, istd_ref, b1_ref,
                 w2_ref, a1_ref, a2_ref, w2s_ref, *, dims):
    (n_pt, d_pt, n_x, d_x, n_z, d_z, n_d, d_d, n_c, L1, L2, RB, D) = dims
    w1 = w1_ref[...]
    r0 = d_pt
    r1 = d_pt + d_x
    r2 = d_pt + d_x + d_z
    r3 = d_pt + d_x + d_z + d_d
    a_pt = jnp.dot(ept_ref[...], w1[0:r0, :],
                   preferred_element_type=jnp.float32) + b1_ref[...]
    a_x = jnp.dot(ex_ref[...], w1[r0:r1, :],
                  preferred_element_type=jnp.float32)
    a_z = jnp.dot(ez_ref[...], w1[r1:r2, :],
                  preferred_element_type=jnp.float32)
    a_d = jnp.dot(ed_ref[...], w1[r2:r3, :],
                  preferred_element_type=jnp.float32)
    a_c = w1[r3:r3 + n_c, :] * istd_ref[...]
    parts1 = [a_pt, a_c]
    if L1 > n_pt + n_c:
        parts1.append(jnp.zeros((L1 - n_pt - n_c, D), jnp.float32))
    a1_ref[...] = jnp.concatenate(parts1, axis=0) * _INV_SQRT2
    pad2 = jnp.zeros((L2 - n_x - n_z - n_d, D), jnp.float32)
    a2_ref[...] = jnp.concatenate([a_x, a_z, a_d, pad2], axis=0) * _INV_SQRT2
    w2s_ref[...] = w2_ref[...] * (0.5 * 2.0 ** 0.5)


def _encoder_kernel(pt_ref, x_ref, z_ref, d_ref, cont_ref,
                    a1_ref, a2_ref, w2_ref, b2_ref,
                    mean_ref, istd_ref, out_ref, *, dims):
    (n_pt, d_pt, n_x, d_x, n_z, d_z, n_d, d_d, n_c, L1, L2, RB, D) = dims
    S = pt_ref.shape[1]

    iota1 = jax.lax.broadcasted_iota(jnp.int32, (RB, S, n_pt), 2)
    oh_pt = (iota1 == pt_ref[...][:, :, None]).astype(jnp.float32)
    cont_n = (cont_ref[...] - mean_ref[0, 0, :]) * istd_ref[0, 0, :]
    aug1 = jnp.concatenate([oh_pt, cont_n], axis=2)

    iota2 = jax.lax.broadcasted_iota(jnp.int32, (RB, S, L2), 2)
    aug2 = ((iota2 == x_ref[...][:, :, None])
            | (iota2 == z_ref[...][:, :, None] + n_x)
            | (iota2 == d_ref[...][:, :, None] + n_x + n_z)
            ).astype(jnp.float32)

    dn = (((2,), (0,)), ((), ()))
    h = jax.lax.dot_general(aug1, a1_ref[...], dn,
                            preferred_element_type=jnp.float32)
    h += jax.lax.dot_general(aug2, a2_ref[...], dn,
                             preferred_element_type=jnp.float32)
    h = h * (1.0 + jax.lax.erf(h))
    out = jax.lax.dot_general(h, w2_ref[...], dn,
                              preferred_element_type=jnp.float32)
    out_ref[...] = out + b2_ref[0, 0, :]


@jax.jit
def kernel(pitch_type_id, x_bin, z_bin, description_id, cont,
           emb_pitch_type, emb_x, emb_z, emb_desc,
           W1, b1, W2, b2, cont_mean, cont_std):
    B, S = pitch_type_id.shape
    n_pt, d_pt = emb_pitch_type.shape
    n_x, d_x = emb_x.shape
    n_z, d_z = emb_z.shape
    n_d, d_d = emb_desc.shape
    n_c = cont.shape[-1]
    D = W2.shape[1]
    L1 = _round_up(n_pt + n_c, 8)
    L2 = _round_up(n_x + n_z + n_d, 8)

    RB = 32
    assert B % RB == 0
    G = B // RB

    istd = 1.0 / jnp.clip(cont_std, 1e-6, None)
    dims = (n_pt, d_pt, n_x, d_x, n_z, d_z, n_d, d_d, n_c, L1, L2, RB, D)
    full = lambda shape: pl.BlockSpec(shape, lambda i: tuple(0 for _ in shape))

    a1, a2, w2s = pl.pallas_call(
        functools.partial(_fold_kernel, dims=dims),
        out_shape=(jax.ShapeDtypeStruct((L1, D), jnp.float32),
                   jax.ShapeDtypeStruct((L2, D), jnp.float32),
                   jax.ShapeDtypeStruct((D, D), jnp.float32)),
    )(emb_pitch_type, emb_x, emb_z, emb_desc, W1, istd.reshape(n_c, 1),
      b1.reshape(1, D), W2)

    out = pl.pallas_call(
        functools.partial(_encoder_kernel, dims=dims),
        grid=(G,),
        in_specs=[
            pl.BlockSpec((RB, S), lambda i: (i, 0)),
            pl.BlockSpec((RB, S), lambda i: (i, 0)),
            pl.BlockSpec((RB, S), lambda i: (i, 0)),
            pl.BlockSpec((RB, S), lambda i: (i, 0)),
            pl.BlockSpec((RB, S, n_c), lambda i: (i, 0, 0)),
            full((L1, D)),
            full((L2, D)),
            full((D, D)),
            pl.BlockSpec((1, 1, D), lambda i: (0, 0, 0)),
            pl.BlockSpec((1, 1, n_c), lambda i: (0, 0, 0)),
            pl.BlockSpec((1, 1, n_c), lambda i: (0, 0, 0)),
        ],
        out_specs=pl.BlockSpec((RB, S, D), lambda i: (i, 0, 0)),
        out_shape=jax.ShapeDtypeStruct((B, S, D), jnp.float32),
        compiler_params=pltpu.CompilerParams(
            dimension_semantics=("parallel",)),
    )(pitch_type_id.astype(jnp.int32), x_bin.astype(jnp.int32),
      z_bin.astype(jnp.int32), description_id.astype(jnp.int32), cont,
      a1, a2, w2s, b2.reshape(1, 1, D),
      cont_mean.reshape(1, 1, n_c), istd.reshape(1, 1, n_c))

    return out
